# baseline (speedup 1.0000x reference)
import functools

import jax
import jax.numpy as jnp
import numpy as np
from jax.experimental import pallas as pl
from jax.experimental.pallas import tpu as pltpu

_KH, _KW = 3, 3


def _deinterleave_matrix(W, C):
    P = np.zeros((C * W, C * (W + 2)), np.float32)
    for c in range(C):
        for wo in range(1, W + 1):
            P[C * (wo - 1) + c, c * (W + 2) + wo] = 1.0
    return jnp.asarray(P)


def _batcher_pairs(n):
    pairs = []

    def oddeven_merge(lo, hi, r):
        step = r * 2
        if step < hi - lo:
            oddeven_merge(lo, hi, step)
            oddeven_merge(lo + r, hi, step)
            for i in range(lo + r, hi - r, step):
                pairs.append((i, i + r))
        else:
            pairs.append((lo, lo + r))

    def sort_range(lo, hi):
        if hi - lo >= 1:
            mid = lo + (hi - lo) // 2
            sort_range(lo, mid)
            sort_range(mid + 1, hi)
            oddeven_merge(lo, hi, 1)

    sort_range(0, n - 1)
    return pairs


def _median27_ops():
    INF = -1
    state = list(range(27)) + [INF] * 5
    next_sym = 27
    raw = []
    for i, j in _batcher_pairs(32):
        a, b = state[i], state[j]
        if a == INF and b == INF:
            continue
        if b == INF:
            continue
        if a == INF:
            state[i], state[j] = b, INF
            continue
        mn, mx = next_sym, next_sym + 1
        next_sym += 2
        raw.append((a, b, mn, mx))
        state[i], state[j] = mn, mx
    out_sym = state[13]
    needed = {out_sym}
    kept = []
    for a, b, mn, mx in reversed(raw):
        need_mn, need_mx = mn in needed, mx in needed
        if not (need_mn or need_mx):
            continue
        needed.discard(mn)
        needed.discard(mx)
        needed.add(a)
        needed.add(b)
        if need_mn and need_mx:
            kept.append(("both", a, b, (mn, mx)))
        elif need_mn:
            kept.append(("min", a, b, mn))
        else:
            kept.append(("max", a, b, mx))
    kept.reverse()
    return kept, out_sym


_MED_OPS, _MED_OUT = _median27_ops()


def _median_kernel(x_ref, med_ref, rsum_ref, *, H, W, C):
    xb = x_ref[0]
    env = {}
    t = 0
    for c in range(C):
        for di in range(_KH):
            for dj in range(_KW):
                env[t] = xb[c, di:di + H, dj:dj + W]
                t += 1
    for kind, a, b, out in _MED_OPS:
        va, vb = env[a], env[b]
        if kind == "both":
            env[out[0]] = jnp.minimum(va, vb)
            env[out[1]] = jnp.maximum(va, vb)
        elif kind == "min":
            env[out] = jnp.minimum(va, vb)
        else:
            env[out] = jnp.maximum(va, vb)
    med = env[_MED_OUT]
    med_ref[0] = med
    rsum_ref[0] = jnp.sum(med, axis=0, keepdims=True)


def _conv_kernel(mu_ref, med_ref, w_ref, out_ref, *, HB, W, F):
    mu = mu_ref[0, 0]
    h0 = pl.multiple_of(pl.program_id(1) * HB, 8)
    m = med_ref[0, pl.ds(h0, HB + 8), :] - mu
    WP = W // 2
    rs = jnp.stack(
        [m[di:di + HB, dj + half * WP:dj + half * WP + WP]
         for half in range(2) for di in range(_KH) for dj in range(_KW)]
    )
    out_ref[0] = jax.lax.dot_general(
        rs, w_ref[...], (((0,), (0,)), ((), ())),
        preferred_element_type=jnp.float32,
    )


def kernel(x, W):
    B, H, Wd, C = x.shape
    F = W.shape[-1]
    HB = 56
    assert H % HB == 0 and C == 3 and W.shape[:3] == (_KH, _KW, 1)

    xt = jnp.transpose(x, (0, 3, 1, 2))
    xp = jnp.pad(xt, ((0, 0), (0, 0), (1, 1), (1, 1)))

    med, rsums = pl.pallas_call(
        functools.partial(_median_kernel, H=H, W=Wd, C=C),
        grid=(B,),
        in_specs=[pl.BlockSpec((1, C, H + 2, Wd + 2), lambda b: (b, 0, 0, 0))],
        out_specs=[
            pl.BlockSpec((1, H, Wd), lambda b: (b, 0, 0)),
            pl.BlockSpec((1, 1, Wd), lambda b: (b, 0, 0)),
        ],
        out_shape=[
            jax.ShapeDtypeStruct((B, H, Wd), jnp.float32),
            jax.ShapeDtypeStruct((B, 1, Wd), jnp.float32),
        ],
        compiler_params=pltpu.CompilerParams(
            dimension_semantics=("parallel",),
        ),
    )(xp)

    mu = jnp.sum(rsums) / (B * H * Wd)
    medp = jnp.pad(med, ((0, 0), (1, 1), (1, 1)), constant_values=mu)
    medp = jnp.pad(medp, ((0, 0), (0, 6), (0, 0)))
    Wv = W.reshape(_KH * _KW, F)
    z = jnp.zeros((_KH * _KW, F), jnp.float32)
    wm = jnp.concatenate(
        [jnp.concatenate([Wv, z], axis=1), jnp.concatenate([z, Wv], axis=1)],
        axis=0)

    out = pl.pallas_call(
        functools.partial(_conv_kernel, HB=HB, W=Wd, F=F),
        grid=(B, H // HB),
        in_specs=[
            pl.BlockSpec(memory_space=pltpu.SMEM),
            pl.BlockSpec((1, H + 8, Wd + 2), lambda b, h: (b, 0, 0)),
            pl.BlockSpec((2 * _KH * _KW, 2 * F), lambda b, h: (0, 0)),
        ],
        out_specs=pl.BlockSpec((1, HB, Wd // 2, 2 * F), lambda b, h: (b, h, 0, 0)),
        out_shape=jax.ShapeDtypeStruct((B, H, Wd // 2, 2 * F), jnp.float32),
        compiler_params=pltpu.CompilerParams(
            dimension_semantics=("parallel", "arbitrary"),
        ),
    )(mu.reshape(1, 1), medp, wm)
    return jnp.concatenate([out[..., :F], out[..., F:]], axis=2)


# speedup vs baseline: 1.6441x; 1.6441x over previous
import functools

import jax
import jax.numpy as jnp
from jax.experimental import pallas as pl
from jax.experimental.pallas import tpu as pltpu

_KH, _KW = 3, 3


def _batcher_pairs(n):
    pairs = []

    def oddeven_merge(lo, hi, r):
        step = r * 2
        if step < hi - lo:
            oddeven_merge(lo, hi, step)
            oddeven_merge(lo + r, hi, step)
            for i in range(lo + r, hi - r, step):
                pairs.append((i, i + r))
        else:
            pairs.append((lo, lo + r))

    def sort_range(lo, hi):
        if hi - lo >= 1:
            mid = lo + (hi - lo) // 2
            sort_range(lo, mid)
            sort_range(mid + 1, hi)
            oddeven_merge(lo, hi, 1)

    sort_range(0, n - 1)
    return pairs


def _median27_ops():
    INF = -1
    state = list(range(27)) + [INF] * 5
    next_sym = 27
    raw = []
    for i, j in _batcher_pairs(32):
        a, b = state[i], state[j]
        if a == INF and b == INF:
            continue
        if b == INF:
            continue
        if a == INF:
            state[i], state[j] = b, INF
            continue
        mn, mx = next_sym, next_sym + 1
        next_sym += 2
        raw.append((a, b, mn, mx))
        state[i], state[j] = mn, mx
    out_sym = state[13]
    needed = {out_sym}
    kept = []
    for a, b, mn, mx in reversed(raw):
        need_mn, need_mx = mn in needed, mx in needed
        if not (need_mn or need_mx):
            continue
        needed.discard(mn)
        needed.discard(mx)
        needed.add(a)
        needed.add(b)
        if need_mn and need_mx:
            kept.append(("both", a, b, (mn, mx)))
        elif need_mn:
            kept.append(("min", a, b, mn))
        else:
            kept.append(("max", a, b, mx))
    kept.reverse()
    return kept, out_sym


_MED_OPS, _MED_OUT = _median27_ops()


def _median_kernel(x_ref, med_ref, rsum_ref, *, H, W, C):
    xb = x_ref[0]
    env = {}
    t = 0
    for c in range(C):
        for di in range(_KH):
            for dj in range(_KW):
                env[t] = xb[c, di:di + H, dj:dj + W]
                t += 1
    for kind, a, b, out in _MED_OPS:
        va, vb = env[a], env[b]
        if kind == "both":
            env[out[0]] = jnp.minimum(va, vb)
            env[out[1]] = jnp.maximum(va, vb)
        elif kind == "min":
            env[out] = jnp.minimum(va, vb)
        else:
            env[out] = jnp.maximum(va, vb)
    med = env[_MED_OUT]
    med_ref[0] = med
    rsum_ref[0] = jnp.sum(med, axis=0, keepdims=True)


def _conv_kernel(mu_ref, med_ref, w_ref, out_ref, *, HB, W, F):
    mu = mu_ref[0, 0]
    h0 = pl.multiple_of(pl.program_id(1) * HB, 8)
    m = med_ref[0, pl.ds(h0, HB + 8), :] - mu
    rs = jnp.stack(
        [m[di:di + HB, dj:dj + W] for di in range(_KH) for dj in range(_KW)]
    )
    out_ref[0] = jax.lax.dot_general(
        rs, w_ref[...], (((0,), (0,)), ((), ())),
        preferred_element_type=jnp.float32,
    )


def kernel(x, W):
    B, H, Wd, C = x.shape
    F = W.shape[-1]
    HB = 56
    assert H % HB == 0 and C == 3 and W.shape[:3] == (_KH, _KW, 1)

    xt = jnp.transpose(x, (0, 3, 1, 2))
    xp = jnp.pad(xt, ((0, 0), (0, 0), (1, 1), (1, 1)))

    med, rsums = pl.pallas_call(
        functools.partial(_median_kernel, H=H, W=Wd, C=C),
        grid=(B,),
        in_specs=[pl.BlockSpec((1, C, H + 2, Wd + 2), lambda b: (b, 0, 0, 0))],
        out_specs=[
            pl.BlockSpec((1, H, Wd), lambda b: (b, 0, 0)),
            pl.BlockSpec((1, 1, Wd), lambda b: (b, 0, 0)),
        ],
        out_shape=[
            jax.ShapeDtypeStruct((B, H, Wd), jnp.float32),
            jax.ShapeDtypeStruct((B, 1, Wd), jnp.float32),
        ],
        compiler_params=pltpu.CompilerParams(
            dimension_semantics=("parallel",),
        ),
    )(xp)

    mu = jnp.sum(rsums) / (B * H * Wd)
    medp = jnp.pad(med, ((0, 0), (1, 1), (1, 1)), constant_values=mu)
    medp = jnp.pad(medp, ((0, 0), (0, 6), (0, 0)))
    wm = W.reshape(_KH * _KW, F)

    out = pl.pallas_call(
        functools.partial(_conv_kernel, HB=HB, W=Wd, F=F),
        grid=(B, H // HB),
        in_specs=[
            pl.BlockSpec(memory_space=pltpu.SMEM),
            pl.BlockSpec((1, H + 8, Wd + 2), lambda b, h: (b, 0, 0)),
            pl.BlockSpec((_KH * _KW, F), lambda b, h: (0, 0)),
        ],
        out_specs=pl.BlockSpec((1, HB, Wd, F), lambda b, h: (b, h, 0, 0)),
        out_shape=jax.ShapeDtypeStruct((B, H, Wd, F), jnp.float32),
        compiler_params=pltpu.CompilerParams(
            dimension_semantics=("parallel", "arbitrary"),
        ),
    )(mu.reshape(1, 1), medp, wm)
    return out
